# baseline (speedup 1.0000x reference)
import jax
import jax.numpy as jnp
import numpy as np
from jax import lax
from jax.experimental import pallas as pl
from jax.experimental.pallas import tpu as pltpu
from jax.experimental.pallas import tpu_sc as plsc

N_SRC = 10000
N_TGT = 10000
E = 320000
IN_DIM = 128
OUT_DIM = 16
N_HEAD = 8
HD = N_HEAD * OUT_DIM
EDGE_EMBED = 16
FFN_HID = 512

NC = 2
NS = 16
EB = 40
E_PER_TILE = E // NS
NBLK = E_PER_TILE // EB
N_ACC = 10240
ROWS_PER_TILE = N_ACC // NS

_r = np.arange(HD)
_r64 = np.arange(64)
_GI = np.zeros((NC, HD, HD), np.float32)
_GI[0, _r64, _r64] = 1.0
_GI[1, 64 + _r64, _r64] = 1.0
_GH = np.zeros((NC, HD, HD), np.float32)
for _c in range(NC):
    for _i in range(HD):
        _j = 64 + _i // OUT_DIM - 4 * _c
        if 64 <= _j < 68:
            _GH[_c, _i, _j] = 1.0
_AH = np.zeros((NC, HD, 16), np.float32)
for _c in range(NC):
    for _i in range(HD):
        _j = _i // OUT_DIM - 4 * _c
        if 0 <= _j < 4:
            _AH[_c, _i, _j] = 1.0
_P0 = np.zeros((HD, HD), np.float32)
_P0[_r64, _r64] = 1.0
_P1 = np.zeros((HD, HD), np.float32)
_P1[_r64, 64 + _r64] = 1.0
_Q0 = np.zeros((HD, HD), np.float32)
_Q0[64 + _r64 // OUT_DIM, _r64] = 1.0
_Q1 = np.zeros((HD, HD), np.float32)
_Q1[64 + _r64 // OUT_DIM, 64 + _r64] = 1.0


def _pre_node_body(x_ref, wfc_ref, a1_ref, gi_ref, gh_ref, zc_ref):
    z = jnp.dot(x_ref[...], wfc_ref[...], preferred_element_type=jnp.float32)
    zc_ref[...] = (
        jnp.dot(z, gi_ref[0], preferred_element_type=jnp.float32)
        + jnp.dot(z * a1_ref[...], gh_ref[0],
                  preferred_element_type=jnp.float32))


def _pre_node(src_h, W_fc, attn1row):
    blk = 1000
    grid = N_SRC // blk
    return pl.pallas_call(
        _pre_node_body,
        grid=(NC, grid),
        in_specs=[
            pl.BlockSpec((blk, IN_DIM), lambda c, i: (i, 0)),
            pl.BlockSpec((IN_DIM, HD), lambda c, i: (0, 0)),
            pl.BlockSpec((1, HD), lambda c, i: (0, 0)),
            pl.BlockSpec((1, HD, HD), lambda c, i: (c, 0, 0)),
            pl.BlockSpec((1, HD, HD), lambda c, i: (c, 0, 0)),
        ],
        out_specs=pl.BlockSpec((blk, HD), lambda c, i: (c * grid + i, 0)),
        out_shape=jax.ShapeDtypeStruct((NC * N_SRC, HD), jnp.float32),
    )(src_h, W_fc, attn1row, jnp.asarray(_GI), jnp.asarray(_GH))


def _pre_edge_body(ea_ref, wf_ref, a3_ref, ah_ref, ef_ref):
    w_e = jnp.dot(wf_ref[...] * a3_ref[...], ah_ref[0],
                  preferred_element_type=jnp.float32)
    ef_ref[...] = jnp.dot(ea_ref[...], w_e, preferred_element_type=jnp.float32)


def _pre_edge(edge_attr, W_feat, attn3row):
    blk = 8000
    grid = E // blk
    return pl.pallas_call(
        _pre_edge_body,
        grid=(NC, grid),
        in_specs=[
            pl.BlockSpec((blk, EDGE_EMBED), lambda c, i: (i, 0)),
            pl.BlockSpec((EDGE_EMBED, HD), lambda c, i: (0, 0)),
            pl.BlockSpec((1, HD), lambda c, i: (0, 0)),
            pl.BlockSpec((1, HD, 16), lambda c, i: (c, 0, 0)),
        ],
        out_specs=pl.BlockSpec((blk, 16), lambda c, i: (c * grid + i, 0)),
        out_shape=jax.ShapeDtypeStruct((NC * E, 16), jnp.float32),
    )(edge_attr, W_feat, attn3row, jnp.asarray(_AH))


def _sc_edge_body(zc_hbm, ef_hbm, src_hbm, dst_hbm, out_hbm,
                  shc, src_v, dst_v, zbuf, efbuf, wbuf, dstcur,
                  semz, semi, sems):
    c = lax.axis_index("c")
    s = lax.axis_index("s")

    zeros16 = jnp.zeros((16,), jnp.float32)
    lane = lax.iota(jnp.int32, 16)
    coff = c * N_SRC

    def _zrow_body(r, _):
        for j in range(HD // 16):
            zbuf[0][r, pl.ds(j * 16, 16)] = zeros16
        return 0

    lax.fori_loop(0, EB, _zrow_body, 0)
    for k in range(ROWS_PER_TILE // EB):
        pltpu.sync_copy(zbuf[0],
                        shc.at[pl.ds(s * ROWS_PER_TILE + k * EB, EB)])
    plsc.subcore_barrier()

    def idx_copies(g, sl):
        base = s * E_PER_TILE + g * EB
        return [
            pltpu.make_async_copy(src_hbm.at[pl.ds(base, EB)],
                                  src_v[sl], semi[sl]),
            pltpu.make_async_copy(dst_hbm.at[pl.ds(base, EB)],
                                  dst_v[sl], semi[sl]),
            pltpu.make_async_copy(ef_hbm.at[pl.ds(c * E + base, EB)],
                                  efbuf[sl], semi[sl]),
        ]

    def idx_start(g, sl):
        for cp in idx_copies(g, sl):
            cp.start()

    def idx_wait(g, sl):
        for cp in idx_copies(g, sl):
            cp.wait()

    def scat_wait(sl):
        pltpu.make_async_copy(zbuf[sl], shc.at[dstcur[sl]], sems[sl]).wait()

    def adjust_and_gather(sl):
        for off in (0, 16):
            src_v[sl][pl.ds(off, 16)] = src_v[sl][pl.ds(off, 16)] + coff
        src_v[sl][pl.ds(24, 16)] = src_v[sl][pl.ds(24, 16)] + jnp.where(
            lane >= 8, coff, 0)
        pltpu.async_copy(zc_hbm.at[src_v[sl]], zbuf[sl], semz[sl])

    idx_start(0, 0)
    idx_wait(0, 0)
    adjust_and_gather(0)
    idx_start(1, 1)

    def _pair_body(i, _):
        for par in range(2):
            g = 2 * i + par
            pltpu.make_async_copy(
                zc_hbm.at[src_v[par]], zbuf[par], semz[par]).wait()
            @pl.when(g + 1 < NBLK)
            def _():
                idx_wait(g + 1, 1 - par)

                def _w():
                    scat_wait(1 - par)

                if par == 0:
                    @pl.when(i >= 1)
                    def _():
                        _w()
                else:
                    _w()
                adjust_and_gather(1 - par)
            for off in (0, 16, 24):
                dstcur[par][pl.ds(off, 16)] = dst_v[par][pl.ds(off, 16)]

            def _msg_body(gg, _):
                for k in range(8):
                    ii = gg * 8 + k
                    x = (zbuf[par][ii, pl.ds(64, 16)]
                         + efbuf[par][ii, pl.ds(0, 16)])
                    w = jnp.where(
                        lane < 4, jnp.exp(jnp.maximum(x, 0.01 * x)), 0.0)
                    for h in range(4):
                        zbuf[par][ii, pl.ds(h * 16, 16)] = (
                            zbuf[par][ii, pl.ds(h * 16, 16)] * w[h])
                    zbuf[par][ii, pl.ds(64, 16)] = w
                return 0

            lax.fori_loop(0, EB // 8, _msg_body, 0)
            @pl.when(g + 2 < NBLK)
            def _():
                idx_start(g + 2, par)
            pltpu.async_copy(
                zbuf[par], shc.at[dstcur[par]], sems[par], add=True)
        return 0

    lax.fori_loop(0, NBLK // 2, _pair_body, 0)
    scat_wait(0)
    scat_wait(1)
    plsc.subcore_barrier()

    r0 = s * ROWS_PER_TILE
    pltpu.sync_copy(shc.at[pl.ds(r0, ROWS_PER_TILE)],
                    out_hbm.at[c, pl.ds(r0, ROWS_PER_TILE)])


def _sc_edge(zcat, efcat, src, dst):
    mesh = plsc.VectorSubcoreMesh(core_axis_name="c", subcore_axis_name="s")
    f32 = jnp.float32
    kern = pl.kernel(
        _sc_edge_body,
        out_type=jax.ShapeDtypeStruct((NC, N_ACC, HD), f32),
        mesh=mesh,
        scratch_types=[
            pltpu.VMEM_SHARED((N_ACC, HD), f32),
            [pltpu.VMEM((EB,), jnp.int32)] * 2,
            [pltpu.VMEM((EB,), jnp.int32)] * 2,
            [pltpu.VMEM((EB, HD), f32)] * 2,
            [pltpu.VMEM((EB, 16), f32)] * 2,
            pltpu.VMEM((EB, 16), f32),
            [pltpu.VMEM((EB,), jnp.int32)] * 2,
            [pltpu.SemaphoreType.DMA] * 2,
            [pltpu.SemaphoreType.DMA] * 2,
            [pltpu.SemaphoreType.DMA] * 2,
        ],
    )
    return kern(zcat, efcat, src, dst)


def _post_body(ha_ref, hb_ref, tgt_ref, p0_ref, p1_ref, q0_ref, q1_ref,
               w1_ref, b1_ref, w2_ref, b2_ref, g_ref, b_ref, y_ref):
    a = ha_ref[...]
    b = hb_ref[...]
    num = (jnp.dot(a, p0_ref[...], preferred_element_type=jnp.float32)
           + jnp.dot(b, p1_ref[...], preferred_element_type=jnp.float32))
    denb = (jnp.dot(a, q0_ref[...], preferred_element_type=jnp.float32)
            + jnp.dot(b, q1_ref[...], preferred_element_type=jnp.float32))
    denb = denb + (denb == 0.0).astype(jnp.float32)
    h = num / denb
    h = jnp.where(h > 0, h, jnp.exp(h) - 1.0) + tgt_ref[...]
    inner = jnp.maximum(
        jnp.dot(h, w1_ref[...], preferred_element_type=jnp.float32)
        + b1_ref[...], 0.0)
    out = (jnp.dot(inner, w2_ref[...], preferred_element_type=jnp.float32)
           + b2_ref[...] + h)
    mean = jnp.mean(out, axis=-1, keepdims=True)
    ctr = out - mean
    var = jnp.mean(ctr * ctr, axis=-1, keepdims=True)
    y_ref[...] = ctr * lax.rsqrt(var + 1e-5) * g_ref[...] + b_ref[...]


def _post(ha, hb, tgt_h, P0, P1, Q0, Q1, W1, b1, W2, b2, ln_g, ln_b):
    blk = 1000
    grid = N_TGT // blk
    full = lambda i: (0, 0)
    return pl.pallas_call(
        _post_body,
        grid=(grid,),
        in_specs=[
            pl.BlockSpec((blk, HD), lambda i: (i, 0)),
            pl.BlockSpec((blk, HD), lambda i: (i, 0)),
            pl.BlockSpec((blk, HD), lambda i: (i, 0)),
            pl.BlockSpec((HD, HD), full),
            pl.BlockSpec((HD, HD), full),
            pl.BlockSpec((HD, HD), full),
            pl.BlockSpec((HD, HD), full),
            pl.BlockSpec((HD, FFN_HID), full),
            pl.BlockSpec((1, FFN_HID), full),
            pl.BlockSpec((FFN_HID, HD), full),
            pl.BlockSpec((1, HD), full),
            pl.BlockSpec((1, HD), full),
            pl.BlockSpec((1, HD), full),
        ],
        out_specs=pl.BlockSpec((blk, HD), lambda i: (i, 0)),
        out_shape=jax.ShapeDtypeStruct((N_TGT, HD), jnp.float32),
    )(ha, hb, tgt_h, P0, P1, Q0, Q1, W1, b1, W2, b2, ln_g, ln_b)


def kernel(src_h, tgt_h, edge_index, edge_attr, W_fc, W_feat, attn,
           W1, b1, W2, b2, ln_g, ln_b):
    f32 = jnp.float32
    src = edge_index[0].astype(jnp.int32)
    dst = edge_index[1].astype(jnp.int32)

    attn1row = attn[0, :, :OUT_DIM].reshape(1, HD).astype(f32)
    attn3row = attn[0, :, 2 * OUT_DIM:].reshape(1, HD).astype(f32)
    P0, P1, Q0, Q1 = (jnp.asarray(_P0), jnp.asarray(_P1),
                      jnp.asarray(_Q0), jnp.asarray(_Q1))

    zcat = _pre_node(src_h, W_fc, attn1row)
    efcat = _pre_edge(edge_attr, W_feat, attn3row)
    outc = _sc_edge(zcat, efcat, src, dst)
    return _post(outc[0, :N_TGT], outc[1, :N_TGT], tgt_h,
                 P0, P1, Q0, Q1,
                 W1, b1.reshape(1, FFN_HID), W2, b2.reshape(1, HD),
                 ln_g.reshape(1, HD), ln_b.reshape(1, HD))


# speedup vs baseline: 1.0277x; 1.0277x over previous
import jax
import jax.numpy as jnp
import numpy as np
from jax import lax
from jax.experimental import pallas as pl
from jax.experimental.pallas import tpu as pltpu
from jax.experimental.pallas import tpu_sc as plsc

N_SRC = 10000
N_TGT = 10000
E = 320000
IN_DIM = 128
OUT_DIM = 16
N_HEAD = 8
HD = N_HEAD * OUT_DIM
EDGE_EMBED = 16
FFN_HID = 512

NC = 2
NS = 16
EB = 40
E_PER_TILE = E // NS
NBLK = E_PER_TILE // EB
N_ACC = 10240
ROWS_PER_TILE = N_ACC // NS

_r = np.arange(HD)
_r64 = np.arange(64)
_GI = np.zeros((NC, HD, HD), np.float32)
_GI[0, _r64, _r64] = 1.0
_GI[1, 64 + _r64, _r64] = 1.0
_GH = np.zeros((NC, HD, HD), np.float32)
for _c in range(NC):
    for _i in range(HD):
        _j = 64 + _i // OUT_DIM - 4 * _c
        if 64 <= _j < 68:
            _GH[_c, _i, _j] = 1.0
_AH = np.zeros((NC, HD, 16), np.float32)
for _c in range(NC):
    for _i in range(HD):
        _j = _i // OUT_DIM - 4 * _c
        if 0 <= _j < 4:
            _AH[_c, _i, _j] = 1.0
_P0 = np.zeros((HD, HD), np.float32)
_P0[_r64, _r64] = 1.0
_P1 = np.zeros((HD, HD), np.float32)
_P1[_r64, 64 + _r64] = 1.0
_Q0 = np.zeros((HD, HD), np.float32)
_Q0[64 + _r64 // OUT_DIM, _r64] = 1.0
_Q1 = np.zeros((HD, HD), np.float32)
_Q1[64 + _r64 // OUT_DIM, 64 + _r64] = 1.0


def _pre_node_body(x_ref, wfc_ref, a1_ref, gi_ref, gh_ref, zc_ref):
    z = jnp.dot(x_ref[...], wfc_ref[...], preferred_element_type=jnp.float32)
    zc_ref[...] = (
        jnp.dot(z, gi_ref[0], preferred_element_type=jnp.float32)
        + jnp.dot(z * a1_ref[...], gh_ref[0],
                  preferred_element_type=jnp.float32))


def _pre_node(src_h, W_fc, attn1row):
    blk = 1000
    grid = N_SRC // blk
    return pl.pallas_call(
        _pre_node_body,
        grid=(NC, grid),
        in_specs=[
            pl.BlockSpec((blk, IN_DIM), lambda c, i: (i, 0)),
            pl.BlockSpec((IN_DIM, HD), lambda c, i: (0, 0)),
            pl.BlockSpec((1, HD), lambda c, i: (0, 0)),
            pl.BlockSpec((1, HD, HD), lambda c, i: (c, 0, 0)),
            pl.BlockSpec((1, HD, HD), lambda c, i: (c, 0, 0)),
        ],
        out_specs=pl.BlockSpec((blk, HD), lambda c, i: (c * grid + i, 0)),
        out_shape=jax.ShapeDtypeStruct((NC * N_SRC, HD), jnp.float32),
    )(src_h, W_fc, attn1row, jnp.asarray(_GI), jnp.asarray(_GH))


def _pre_edge_body(ea_ref, wf_ref, a3_ref, ah_ref, ef_ref):
    w_e = jnp.dot(wf_ref[...] * a3_ref[...], ah_ref[0],
                  preferred_element_type=jnp.float32)
    ef_ref[...] = jnp.dot(ea_ref[...], w_e, preferred_element_type=jnp.float32)


def _pre_edge(edge_attr, W_feat, attn3row):
    blk = 8000
    grid = E // blk
    return pl.pallas_call(
        _pre_edge_body,
        grid=(NC, grid),
        in_specs=[
            pl.BlockSpec((blk, EDGE_EMBED), lambda c, i: (i, 0)),
            pl.BlockSpec((EDGE_EMBED, HD), lambda c, i: (0, 0)),
            pl.BlockSpec((1, HD), lambda c, i: (0, 0)),
            pl.BlockSpec((1, HD, 16), lambda c, i: (c, 0, 0)),
        ],
        out_specs=pl.BlockSpec((blk, 16), lambda c, i: (c * grid + i, 0)),
        out_shape=jax.ShapeDtypeStruct((NC * E, 16), jnp.float32),
    )(edge_attr, W_feat, attn3row, jnp.asarray(_AH))


def _sc_edge_body(zc_hbm, ef_hbm, src_hbm, dst_hbm, out_hbm,
                  shc, src_v, dst_v, zbuf, efbuf, wbuf, dstcur,
                  semz, semi, sems):
    c = lax.axis_index("c")
    s = lax.axis_index("s")

    zeros16 = jnp.zeros((16,), jnp.float32)
    lane = lax.iota(jnp.int32, 16)
    coff = c * N_SRC

    def _zrow_body(r, _):
        for j in range(HD // 16):
            zbuf[0][r, pl.ds(j * 16, 16)] = zeros16
        return 0

    lax.fori_loop(0, EB, _zrow_body, 0)
    for k in range(ROWS_PER_TILE // EB):
        pltpu.sync_copy(zbuf[0],
                        shc.at[pl.ds(s * ROWS_PER_TILE + k * EB, EB)])
    plsc.subcore_barrier()

    def idx_copies(g, sl):
        base = s * E_PER_TILE + g * EB
        return [
            pltpu.make_async_copy(src_hbm.at[pl.ds(base, EB)],
                                  src_v[sl], semi[sl]),
            pltpu.make_async_copy(dst_hbm.at[pl.ds(base, EB)],
                                  dst_v[sl], semi[sl]),
            pltpu.make_async_copy(ef_hbm.at[pl.ds(c * E + base, EB)],
                                  efbuf[sl], semi[sl]),
        ]

    def idx_start(g, sl):
        for cp in idx_copies(g, sl):
            cp.start()

    def idx_wait(g, sl):
        for cp in idx_copies(g, sl):
            cp.wait()

    def scat_wait(sl):
        pltpu.make_async_copy(zbuf[sl], shc.at[dstcur[sl]], sems[sl]).wait()

    def adjust_and_gather(sl):
        for off in (0, 16):
            src_v[sl][pl.ds(off, 16)] = src_v[sl][pl.ds(off, 16)] + coff
        src_v[sl][pl.ds(24, 16)] = src_v[sl][pl.ds(24, 16)] + jnp.where(
            lane >= 8, coff, 0)
        pltpu.async_copy(zc_hbm.at[src_v[sl]], zbuf[sl], semz[sl])

    idx_start(0, 0)
    idx_wait(0, 0)
    adjust_and_gather(0)
    idx_start(1, 1)

    def _pair_body(i, _):
        for par in range(2):
            g = 2 * i + par
            pltpu.make_async_copy(
                zc_hbm.at[src_v[par]], zbuf[par], semz[par]).wait()
            @pl.when(g + 1 < NBLK)
            def _():
                idx_wait(g + 1, 1 - par)

                def _w():
                    scat_wait(1 - par)

                if par == 0:
                    @pl.when(i >= 1)
                    def _():
                        _w()
                else:
                    _w()
                adjust_and_gather(1 - par)
            def _w_body(gg, _):
                i8 = gg * 8
                for k in range(8):
                    x = (zbuf[par][i8 + k, pl.ds(64, 16)]
                         + efbuf[par][i8 + k, pl.ds(0, 16)])
                    wbuf[i8 + k, pl.ds(0, 16)] = jnp.where(
                        lane < 4, jnp.exp(jnp.maximum(x, 0.01 * x)), 0.0)
                return 0

            lax.fori_loop(0, EB // 8, _w_body, 0)
            for off in (0, 16, 24):
                dstcur[par][pl.ds(off, 16)] = dst_v[par][pl.ds(off, 16)]
            @pl.when(g + 2 < NBLK)
            def _():
                idx_start(g + 2, par)

            def _msg_body(gg, _):
                for k in range(8):
                    ii = gg * 8 + k
                    w = wbuf[ii, pl.ds(0, 16)]
                    for h in range(4):
                        zbuf[par][ii, pl.ds(h * 16, 16)] = (
                            zbuf[par][ii, pl.ds(h * 16, 16)] * w[h])
                    zbuf[par][ii, pl.ds(64, 16)] = w
                return 0

            lax.fori_loop(0, EB // 8, _msg_body, 0)
            pltpu.async_copy(
                zbuf[par], shc.at[dstcur[par]], sems[par], add=True)
        return 0

    lax.fori_loop(0, NBLK // 2, _pair_body, 0)
    scat_wait(0)
    scat_wait(1)
    plsc.subcore_barrier()

    r0 = s * ROWS_PER_TILE
    pltpu.sync_copy(shc.at[pl.ds(r0, ROWS_PER_TILE)],
                    out_hbm.at[c, pl.ds(r0, ROWS_PER_TILE)])


def _sc_edge(zcat, efcat, src, dst):
    mesh = plsc.VectorSubcoreMesh(core_axis_name="c", subcore_axis_name="s")
    f32 = jnp.float32
    kern = pl.kernel(
        _sc_edge_body,
        out_type=jax.ShapeDtypeStruct((NC, N_ACC, HD), f32),
        mesh=mesh,
        scratch_types=[
            pltpu.VMEM_SHARED((N_ACC, HD), f32),
            [pltpu.VMEM((EB,), jnp.int32)] * 2,
            [pltpu.VMEM((EB,), jnp.int32)] * 2,
            [pltpu.VMEM((EB, HD), f32)] * 2,
            [pltpu.VMEM((EB, 16), f32)] * 2,
            pltpu.VMEM((EB, 16), f32),
            [pltpu.VMEM((EB,), jnp.int32)] * 2,
            [pltpu.SemaphoreType.DMA] * 2,
            [pltpu.SemaphoreType.DMA] * 2,
            [pltpu.SemaphoreType.DMA] * 2,
        ],
    )
    return kern(zcat, efcat, src, dst)


def _post_body(ha_ref, hb_ref, tgt_ref, p0_ref, p1_ref, q0_ref, q1_ref,
               w1_ref, b1_ref, w2_ref, b2_ref, g_ref, b_ref, y_ref):
    a = ha_ref[...]
    b = hb_ref[...]
    num = (jnp.dot(a, p0_ref[...], preferred_element_type=jnp.float32)
           + jnp.dot(b, p1_ref[...], preferred_element_type=jnp.float32))
    denb = (jnp.dot(a, q0_ref[...], preferred_element_type=jnp.float32)
            + jnp.dot(b, q1_ref[...], preferred_element_type=jnp.float32))
    denb = denb + (denb == 0.0).astype(jnp.float32)
    h = num / denb
    h = jnp.where(h > 0, h, jnp.exp(h) - 1.0) + tgt_ref[...]
    inner = jnp.maximum(
        jnp.dot(h, w1_ref[...], preferred_element_type=jnp.float32)
        + b1_ref[...], 0.0)
    out = (jnp.dot(inner, w2_ref[...], preferred_element_type=jnp.float32)
           + b2_ref[...] + h)
    mean = jnp.mean(out, axis=-1, keepdims=True)
    ctr = out - mean
    var = jnp.mean(ctr * ctr, axis=-1, keepdims=True)
    y_ref[...] = ctr * lax.rsqrt(var + 1e-5) * g_ref[...] + b_ref[...]


def _post(ha, hb, tgt_h, P0, P1, Q0, Q1, W1, b1, W2, b2, ln_g, ln_b):
    blk = 1000
    grid = N_TGT // blk
    full = lambda i: (0, 0)
    return pl.pallas_call(
        _post_body,
        grid=(grid,),
        in_specs=[
            pl.BlockSpec((blk, HD), lambda i: (i, 0)),
            pl.BlockSpec((blk, HD), lambda i: (i, 0)),
            pl.BlockSpec((blk, HD), lambda i: (i, 0)),
            pl.BlockSpec((HD, HD), full),
            pl.BlockSpec((HD, HD), full),
            pl.BlockSpec((HD, HD), full),
            pl.BlockSpec((HD, HD), full),
            pl.BlockSpec((HD, FFN_HID), full),
            pl.BlockSpec((1, FFN_HID), full),
            pl.BlockSpec((FFN_HID, HD), full),
            pl.BlockSpec((1, HD), full),
            pl.BlockSpec((1, HD), full),
            pl.BlockSpec((1, HD), full),
        ],
        out_specs=pl.BlockSpec((blk, HD), lambda i: (i, 0)),
        out_shape=jax.ShapeDtypeStruct((N_TGT, HD), jnp.float32),
    )(ha, hb, tgt_h, P0, P1, Q0, Q1, W1, b1, W2, b2, ln_g, ln_b)


def kernel(src_h, tgt_h, edge_index, edge_attr, W_fc, W_feat, attn,
           W1, b1, W2, b2, ln_g, ln_b):
    f32 = jnp.float32
    src = edge_index[0].astype(jnp.int32)
    dst = edge_index[1].astype(jnp.int32)

    attn1row = attn[0, :, :OUT_DIM].reshape(1, HD).astype(f32)
    attn3row = attn[0, :, 2 * OUT_DIM:].reshape(1, HD).astype(f32)
    P0, P1, Q0, Q1 = (jnp.asarray(_P0), jnp.asarray(_P1),
                      jnp.asarray(_Q0), jnp.asarray(_Q1))

    zcat = _pre_node(src_h, W_fc, attn1row)
    efcat = _pre_edge(edge_attr, W_feat, attn3row)
    outc = _sc_edge(zcat, efcat, src, dst)
    return _post(outc[0, :N_TGT], outc[1, :N_TGT], tgt_h,
                 P0, P1, Q0, Q1,
                 W1, b1.reshape(1, FFN_HID), W2, b2.reshape(1, HD),
                 ln_g.reshape(1, HD), ln_b.reshape(1, HD))


# speedup vs baseline: 1.0352x; 1.0073x over previous
import jax
import jax.numpy as jnp
import numpy as np
from jax import lax
from jax.experimental import pallas as pl
from jax.experimental.pallas import tpu as pltpu
from jax.experimental.pallas import tpu_sc as plsc

N_SRC = 10000
N_TGT = 10000
E = 320000
IN_DIM = 128
OUT_DIM = 16
N_HEAD = 8
HD = N_HEAD * OUT_DIM
EDGE_EMBED = 16
FFN_HID = 512

NC = 2
NS = 16
EB = 40
E_PER_TILE = E // NS
NBLK = E_PER_TILE // EB
N_ACC = 10240
ROWS_PER_TILE = N_ACC // NS

_r = np.arange(HD)
_r64 = np.arange(64)
_GI = np.zeros((NC, HD, HD), np.float32)
_GI[0, _r64, _r64] = 1.0
_GI[1, 64 + _r64, _r64] = 1.0
_GH = np.zeros((NC, HD, HD), np.float32)
for _c in range(NC):
    for _i in range(HD):
        _j = 64 + _i // OUT_DIM - 4 * _c
        if 64 <= _j < 68:
            _GH[_c, _i, _j] = 1.0
_AH = np.zeros((NC, HD, 16), np.float32)
for _c in range(NC):
    for _i in range(HD):
        _j = _i // OUT_DIM - 4 * _c
        if 0 <= _j < 4:
            _AH[_c, _i, _j] = 1.0
_P0 = np.zeros((HD, HD), np.float32)
_P0[_r64, _r64] = 1.0
_P1 = np.zeros((HD, HD), np.float32)
_P1[_r64, 64 + _r64] = 1.0
_Q0 = np.zeros((HD, HD), np.float32)
_Q0[64 + _r64 // OUT_DIM, _r64] = 1.0
_Q1 = np.zeros((HD, HD), np.float32)
_Q1[64 + _r64 // OUT_DIM, 64 + _r64] = 1.0


def _pre_node_body(x_ref, wfc_ref, a1_ref, gi_ref, gh_ref, zc_ref):
    z = jnp.dot(x_ref[...], wfc_ref[...], preferred_element_type=jnp.float32)
    zc_ref[...] = (
        jnp.dot(z, gi_ref[0], preferred_element_type=jnp.float32)
        + jnp.dot(z * a1_ref[...], gh_ref[0],
                  preferred_element_type=jnp.float32))


def _pre_node(src_h, W_fc, attn1row):
    blk = 1000
    grid = N_SRC // blk
    return pl.pallas_call(
        _pre_node_body,
        grid=(NC, grid),
        in_specs=[
            pl.BlockSpec((blk, IN_DIM), lambda c, i: (i, 0)),
            pl.BlockSpec((IN_DIM, HD), lambda c, i: (0, 0)),
            pl.BlockSpec((1, HD), lambda c, i: (0, 0)),
            pl.BlockSpec((1, HD, HD), lambda c, i: (c, 0, 0)),
            pl.BlockSpec((1, HD, HD), lambda c, i: (c, 0, 0)),
        ],
        out_specs=pl.BlockSpec((blk, HD), lambda c, i: (c * grid + i, 0)),
        out_shape=jax.ShapeDtypeStruct((NC * N_SRC, HD), jnp.float32),
    )(src_h, W_fc, attn1row, jnp.asarray(_GI), jnp.asarray(_GH))


def _pre_edge_body(ea_ref, wf_ref, a3_ref, ah_ref, ef_ref):
    w_e = jnp.dot(wf_ref[...] * a3_ref[...], ah_ref[0],
                  preferred_element_type=jnp.float32)
    ef_ref[...] = jnp.dot(ea_ref[...], w_e, preferred_element_type=jnp.float32)


def _pre_edge(edge_attr, W_feat, attn3row):
    blk = 8000
    grid = E // blk
    return pl.pallas_call(
        _pre_edge_body,
        grid=(NC, grid),
        in_specs=[
            pl.BlockSpec((blk, EDGE_EMBED), lambda c, i: (i, 0)),
            pl.BlockSpec((EDGE_EMBED, HD), lambda c, i: (0, 0)),
            pl.BlockSpec((1, HD), lambda c, i: (0, 0)),
            pl.BlockSpec((1, HD, 16), lambda c, i: (c, 0, 0)),
        ],
        out_specs=pl.BlockSpec((blk, 16), lambda c, i: (c * grid + i, 0)),
        out_shape=jax.ShapeDtypeStruct((NC * E, 16), jnp.float32),
    )(edge_attr, W_feat, attn3row, jnp.asarray(_AH))


def _sc_edge_body(zc_hbm, ef_hbm, src_hbm, dst_hbm, out_hbm,
                  shc, src_v, dst_v, zbuf, efbuf, wbuf, dstcur,
                  semz, semi, sems):
    c = lax.axis_index("c")
    s = lax.axis_index("s")

    zeros16 = jnp.zeros((16,), jnp.float32)
    lane = lax.iota(jnp.int32, 16)
    coff = c * N_SRC

    def _zrow_body(r, _):
        for j in range(HD // 16):
            zbuf[0][r, pl.ds(j * 16, 16)] = zeros16
        return 0

    lax.fori_loop(0, EB, _zrow_body, 0)
    for k in range(ROWS_PER_TILE // EB):
        pltpu.sync_copy(zbuf[0],
                        shc.at[pl.ds(s * ROWS_PER_TILE + k * EB, EB)])
    plsc.subcore_barrier()

    def idx_copies(g, sl):
        base = s * E_PER_TILE + g * EB
        return [
            pltpu.make_async_copy(src_hbm.at[pl.ds(base, EB)],
                                  src_v[sl], semi[sl]),
            pltpu.make_async_copy(dst_hbm.at[pl.ds(base, EB)],
                                  dst_v[sl], semi[sl]),
            pltpu.make_async_copy(ef_hbm.at[pl.ds(c * E + base, EB)],
                                  efbuf[sl], semi[sl]),
        ]

    def idx_start(g, sl):
        for cp in idx_copies(g, sl):
            cp.start()

    def idx_wait(g, sl):
        for cp in idx_copies(g, sl):
            cp.wait()

    def scat_wait(sl):
        pltpu.make_async_copy(zbuf[sl], shc.at[dstcur[sl]], sems[sl]).wait()

    def adjust_and_gather(sl):
        for off in (0, 16):
            src_v[sl][pl.ds(off, 16)] = src_v[sl][pl.ds(off, 16)] + coff
        src_v[sl][pl.ds(24, 16)] = src_v[sl][pl.ds(24, 16)] + jnp.where(
            lane >= 8, coff, 0)
        pltpu.async_copy(zc_hbm.at[src_v[sl]], zbuf[sl], semz[sl])

    idx_start(0, 0)
    idx_wait(0, 0)
    adjust_and_gather(0)
    idx_start(1, 1)

    def _pair_body(i, _):
        for par in range(2):
            g = 2 * i + par
            pltpu.make_async_copy(
                zc_hbm.at[src_v[par]], zbuf[par], semz[par]).wait()
            @pl.when(g + 1 < NBLK)
            def _():
                idx_wait(g + 1, 1 - par)

                def _w():
                    scat_wait(1 - par)

                if par == 0:
                    @pl.when(i >= 1)
                    def _():
                        _w()
                else:
                    _w()
                adjust_and_gather(1 - par)
            def _w_body(gg, _):
                i8 = gg * 8
                for k in range(8):
                    x = (zbuf[par][i8 + k, pl.ds(64, 16)]
                         + efbuf[par][i8 + k, pl.ds(0, 16)])
                    wbuf[i8 + k, pl.ds(0, 16)] = jnp.where(
                        lane < 4, jnp.exp(jnp.maximum(x, 0.01 * x)), 0.0)
                return 0

            lax.fori_loop(0, EB // 8, _w_body, 0)
            for off in (0, 16, 24):
                dstcur[par][pl.ds(off, 16)] = dst_v[par][pl.ds(off, 16)]
            @pl.when(g + 2 < NBLK)
            def _():
                idx_start(g + 2, par)

            def _msg_body(gg, _):
                for k in range(8):
                    ii = gg * 8 + k
                    w = wbuf[ii, pl.ds(0, 16)]
                    for h in range(4):
                        zbuf[par][ii, pl.ds(h * 16, 16)] = (
                            zbuf[par][ii, pl.ds(h * 16, 16)] * w[h])
                    zbuf[par][ii, pl.ds(64, 16)] = w
                return 0

            lax.fori_loop(0, EB // 8, _msg_body, 0)
            pltpu.async_copy(
                zbuf[par], shc.at[dstcur[par]], sems[par], add=True)
        return 0

    lax.fori_loop(0, NBLK // 2, _pair_body, 0)
    scat_wait(0)
    scat_wait(1)
    plsc.subcore_barrier()

    r0 = s * ROWS_PER_TILE
    pltpu.sync_copy(shc.at[pl.ds(r0, ROWS_PER_TILE)],
                    out_hbm.at[c, pl.ds(r0, ROWS_PER_TILE)])


def _sc_edge(zcat, efcat, src, dst):
    mesh = plsc.VectorSubcoreMesh(core_axis_name="c", subcore_axis_name="s")
    f32 = jnp.float32
    kern = pl.kernel(
        _sc_edge_body,
        out_type=jax.ShapeDtypeStruct((NC, N_ACC, HD), f32),
        mesh=mesh,
        scratch_types=[
            pltpu.VMEM_SHARED((N_ACC, HD), f32),
            [pltpu.VMEM((EB,), jnp.int32)] * 2,
            [pltpu.VMEM((EB,), jnp.int32)] * 2,
            [pltpu.VMEM((EB, HD), f32)] * 2,
            [pltpu.VMEM((EB, 16), f32)] * 2,
            pltpu.VMEM((EB, 16), f32),
            [pltpu.VMEM((EB,), jnp.int32)] * 2,
            [pltpu.SemaphoreType.DMA] * 2,
            [pltpu.SemaphoreType.DMA] * 2,
            [pltpu.SemaphoreType.DMA] * 2,
        ],
    )
    return kern(zcat, efcat, src, dst)


def _post_body(hc_ref, tgt_ref, p0_ref, p1_ref, q0_ref, q1_ref,
               w1_ref, b1_ref, w2_ref, b2_ref, g_ref, b_ref, y_ref):
    a = hc_ref[0]
    b = hc_ref[1]
    num = (jnp.dot(a, p0_ref[...], preferred_element_type=jnp.float32)
           + jnp.dot(b, p1_ref[...], preferred_element_type=jnp.float32))
    denb = (jnp.dot(a, q0_ref[...], preferred_element_type=jnp.float32)
            + jnp.dot(b, q1_ref[...], preferred_element_type=jnp.float32))
    denb = denb + (denb == 0.0).astype(jnp.float32)
    h = num / denb
    h = jnp.where(h > 0, h, jnp.exp(h) - 1.0) + tgt_ref[...]
    inner = jnp.maximum(
        jnp.dot(h, w1_ref[...], preferred_element_type=jnp.float32)
        + b1_ref[...], 0.0)
    out = (jnp.dot(inner, w2_ref[...], preferred_element_type=jnp.float32)
           + b2_ref[...] + h)
    mean = jnp.mean(out, axis=-1, keepdims=True)
    ctr = out - mean
    var = jnp.mean(ctr * ctr, axis=-1, keepdims=True)
    y_ref[...] = ctr * lax.rsqrt(var + 1e-5) * g_ref[...] + b_ref[...]


def _post(outc, tgt_h, P0, P1, Q0, Q1, W1, b1, W2, b2, ln_g, ln_b):
    blk = 1000
    grid = N_TGT // blk
    full = lambda i: (0, 0)
    return pl.pallas_call(
        _post_body,
        grid=(grid,),
        in_specs=[
            pl.BlockSpec((NC, blk, HD), lambda i: (0, i, 0)),
            pl.BlockSpec((blk, HD), lambda i: (i, 0)),
            pl.BlockSpec((HD, HD), full),
            pl.BlockSpec((HD, HD), full),
            pl.BlockSpec((HD, HD), full),
            pl.BlockSpec((HD, HD), full),
            pl.BlockSpec((HD, FFN_HID), full),
            pl.BlockSpec((1, FFN_HID), full),
            pl.BlockSpec((FFN_HID, HD), full),
            pl.BlockSpec((1, HD), full),
            pl.BlockSpec((1, HD), full),
            pl.BlockSpec((1, HD), full),
        ],
        out_specs=pl.BlockSpec((blk, HD), lambda i: (i, 0)),
        out_shape=jax.ShapeDtypeStruct((N_TGT, HD), jnp.float32),
    )(outc, tgt_h, P0, P1, Q0, Q1, W1, b1, W2, b2, ln_g, ln_b)


def kernel(src_h, tgt_h, edge_index, edge_attr, W_fc, W_feat, attn,
           W1, b1, W2, b2, ln_g, ln_b):
    f32 = jnp.float32
    ei = edge_index
    if ei.dtype != jnp.int32:
        ei = ei.astype(jnp.int32)
    src = ei[0]
    dst = ei[1]

    attn1row = attn[0, :, :OUT_DIM].reshape(1, HD).astype(f32)
    attn3row = attn[0, :, 2 * OUT_DIM:].reshape(1, HD).astype(f32)
    P0, P1, Q0, Q1 = (jnp.asarray(_P0), jnp.asarray(_P1),
                      jnp.asarray(_Q0), jnp.asarray(_Q1))

    zcat = _pre_node(src_h, W_fc, attn1row)
    efcat = _pre_edge(edge_attr, W_feat, attn3row)
    outc = _sc_edge(zcat, efcat, src, dst)
    return _post(outc, tgt_h,
                 P0, P1, Q0, Q1,
                 W1, b1.reshape(1, FFN_HID), W2, b2.reshape(1, HD),
                 ln_g.reshape(1, HD), ln_b.reshape(1, HD))
